# Initial kernel scaffold; baseline (speedup 1.0000x reference)
import jax
import jax.numpy as jnp
from jax.experimental import pallas as pl


def kernel(g, params):
    raise NotImplementedError("write your pallas kernel here")


# speedup vs baseline: 1.4182x; 1.4182x over previous
import jax
import jax.numpy as jnp
from jax import lax
from jax.experimental import pallas as pl
from jax.experimental.pallas import tpu as pltpu

_ROI = 1500
_N = 1536
_S = 128
_NS = _N // _S
_KS = (1200, 900, 600, 300)

_DW, _DB, _UW, _UB, _PW, _PB = 0, 4, 8, 12, 16, 20
_SB, _BW, _BB, _EW0, _EW1, _EB = 24, 25, 26, 27, 28, 29

_F32 = jnp.float32
_BF16 = jnp.bfloat16
_CN = (((1,), (0,)), ((), ()))


def _unet_body(g_ref, sw_ref, sc_ref, out_ref,
               b0, b1, b2, b3, rank_ref, rs_ref, s_sc, k_sc, v_sc, mv_ref):
    bmats = (b0, b1, b2, b3, b0)

    def sc(i):
        return sc_ref[i]

    def t(x):
        return jnp.transpose(x)

    def mv_g(x):
        def strip(i, c):
            left = g_ref[0, pl.ds(i * _S, _S), :]
            mv_ref[pl.ds(i * _S, _S), :] = lax.dot_general(
                left, x, _CN, precision=lax.Precision.HIGHEST,
                preferred_element_type=_F32)
            return c

        lax.fori_loop(0, _NS, strip, 0)
        return mv_ref[...]

    def mv_b(bref, x):
        a = x.astype(_BF16)
        r = x - a.astype(_F32)
        b_ = r.astype(_BF16)
        c = (r - b_.astype(_F32)).astype(_BF16)

        def strip(i, cc):
            left = bref[pl.ds(i * _S, _S), :]
            acc = lax.dot_general(left, a, _CN, preferred_element_type=_F32)
            acc += lax.dot_general(left, b_, _CN, preferred_element_type=_F32)
            acc += lax.dot_general(left, c, _CN, preferred_element_type=_F32)
            mv_ref[pl.ds(i * _S, _S), :] = acc
            return cc

        lax.fori_loop(0, _NS, strip, 0)
        return mv_ref[...]

    def sigmoid(x):
        e = jnp.exp(-jnp.abs(x))
        return jnp.where(x >= 0, 1.0 / (1.0 + e), e / (1.0 + e))

    def rank_topk(s, key, valid, k):
        srow, krow = t(s), t(key)
        vrow = t(valid) > 0.0
        s_sc[...] = s
        k_sc[...] = key

        def strip(i, carry):
            sc_ = s_sc[pl.ds(i * _S, _S), :]
            kc_ = k_sc[pl.ds(i * _S, _S), :]
            beats = ((srow > sc_) | ((srow == sc_) & (krow < kc_))) & vrow
            rank_ref[pl.ds(i * _S, _S), :] = jnp.sum(
                jnp.where(beats, 1.0, 0.0), axis=1, keepdims=True)
            return carry

        lax.fori_loop(0, _NS, strip, 0)
        rank = rank_ref[...]
        sel = jnp.where((valid > 0.0) & (rank < float(k)), 1.0, 0.0)
        return sel, rank

    def pool_graph(bsrc, bdst, sel):
        selrow = t(sel)
        v_sc[...] = sel
        bfull = bsrc[...]

        def strip(i, carry):
            left = bsrc[pl.ds(i * _S, _S), :]
            m = lax.dot_general(left, bfull, _CN,
                                preferred_element_type=_F32)
            selc = v_sc[pl.ds(i * _S, _S), :]
            bn = jnp.where((m > 0.0) & (selc > 0.0) & (selrow > 0.0),
                           1.0, 0.0)
            bdst[pl.ds(i * _S, _S), :] = bn.astype(_BF16)
            rs_ref[pl.ds(i * _S, _S), :] = jnp.sum(bn, axis=1, keepdims=True)
            return carry

        lax.fori_loop(0, _NS, strip, 0)
        return 1.0 / (rs_ref[...] + 1e-8)

    iota_col = lax.broadcasted_iota(jnp.int32, (_N, 1), 0)
    valid0 = jnp.where(iota_col < _ROI, 1.0, 0.0)

    def strip_b0(i, carry):
        b0[pl.ds(i * _S, _S), :] = (
            g_ref[0, pl.ds(i * _S, _S), :] != 0.0).astype(_BF16)
        return carry

    lax.fori_loop(0, _NS, strip_b0, 0)

    h0 = jnp.maximum(mv_g(sw_ref[...]) + sc(_SB), 0.0) * valid0
    org_h = h0

    masks = [valid0]
    invss = [None]
    downs = []
    h = h0
    key = iota_col.astype(_F32)
    for lvl in range(4):
        if lvl == 0:
            tv = mv_g(h)
        else:
            tv = mv_b(bmats[lvl], h) * invss[lvl]
        h = jnp.maximum(tv * sc(_DW + lvl) + sc(_DB + lvl), 0.0) * masks[lvl]
        downs.append(h)
        s = sigmoid(h * sc(_PW + lvl) + sc(_PB + lvl))
        sel, rank = rank_topk(s, key, masks[lvl], _KS[lvl])
        h = h * s * sel
        key = rank
        invss.append(pool_graph(bmats[lvl], bmats[lvl + 1], sel))
        masks.append(sel)

    tv = mv_b(bmats[4], h) * invss[4]
    h = jnp.maximum(tv * sc(_BW) + sc(_BB), 0.0) * masks[4]

    for i in range(4):
        up = 3 - i
        if up == 0:
            tv = mv_g(h)
        else:
            tv = mv_b(bmats[up], h) * invss[up]
        h = jnp.maximum(tv * sc(_UW + i) + sc(_UB + i), 0.0) * masks[up]
        h = h + downs[up]

    acc = mv_g(h) * sc(_EW0)
    acc = acc + mv_g(org_h) * sc(_EW1)
    res = jnp.maximum(acc + sc(_EB), 0.0) * valid0
    out_ref[0] = t(res)


def _mlp_body(x_ref, g1, b1, w1, c1, g2, b2, w2, c2,
              g3, b3, w3, c3, g4, b4, w4, c4, o_ref):
    def bn(x, ga, be):
        m = jnp.mean(x, axis=0, keepdims=True)
        v = jnp.mean((x - m) ** 2, axis=0, keepdims=True)
        return (x - m) / jnp.sqrt(v + 1e-5) * ga[...] + be[...]

    def dot(a, b):
        return lax.dot_general(a, b, _CN, precision=lax.Precision.HIGHEST,
                               preferred_element_type=_F32)

    h = x_ref[...]
    for ga, be, w, c in ((g1, b1, w1, c1), (g2, b2, w2, c2),
                         (g3, b3, w3, c3), (g4, b4, w4, c4)):
        h = jnp.maximum(bn(h, ga, be), 0.0)
        h = dot(h, w[...]) + c[...]
    z = h - jnp.max(h, axis=1, keepdims=True)
    e = jnp.exp(z)
    o_ref[...] = e / jnp.sum(e, axis=1, keepdims=True)


def kernel(g, params):
    b = g.shape[0]
    pad = _N - _ROI
    gp = jnp.pad(g.astype(_F32), ((0, 0), (0, pad), (0, pad)))
    swp = jnp.pad(params['start_w'].astype(_F32), ((0, pad), (0, 0)))

    vals = []
    for key in ('down_w', 'down_b', 'up_w', 'up_b', 'pool_w', 'pool_b'):
        vals += [params[key][i].reshape(()) for i in range(4)]
    vals += [params['start_b'].reshape(()),
             params['bottom_w'].reshape(()), params['bottom_b'].reshape(()),
             params['end_w'][0, 0], params['end_w'][1, 0],
             params['end_b'].reshape(())]
    scal = jnp.stack([v.astype(_F32) for v in vals])

    unet_out = pl.pallas_call(
        _unet_body,
        grid=(b,),
        in_specs=[
            pl.BlockSpec((1, _N, _N), lambda i: (i, 0, 0),
                         pipeline_mode=pl.Buffered(buffer_count=1)),
            pl.BlockSpec((_N, 1), lambda i: (0, 0)),
            pl.BlockSpec(memory_space=pltpu.SMEM),
        ],
        out_specs=pl.BlockSpec((1, 1, _N), lambda i: (i, 0, 0)),
        out_shape=jax.ShapeDtypeStruct((b, 1, _N), _F32),
        scratch_shapes=[pltpu.VMEM((_N, _N), _BF16)] * 4
        + [pltpu.VMEM((_N, 1), _F32)] * 6,
    )(gp, swp, scal)
    unet_out = unet_out.reshape(b, _N)

    w1p = jnp.pad(params['fl1_w'].astype(_F32), ((0, pad), (0, 0)))
    g1p = jnp.pad(params['bn1_g'].astype(_F32), (0, pad)).reshape(1, _N)
    b1p = jnp.pad(params['bn1_b'].astype(_F32), (0, pad)).reshape(1, _N)

    args = [unet_out,
            g1p, b1p, w1p, params['fl1_b'].reshape(1, -1),
            params['bn2_g'].reshape(1, -1), params['bn2_b'].reshape(1, -1),
            params['fl2_w'], params['fl2_b'].reshape(1, -1),
            params['bn3_g'].reshape(1, -1), params['bn3_b'].reshape(1, -1),
            params['fl3_w'], params['fl3_b'].reshape(1, -1),
            params['bn4_g'].reshape(1, -1), params['bn4_b'].reshape(1, -1),
            params['fl4_w'], params['fl4_b'].reshape(1, -1)]

    out = pl.pallas_call(
        _mlp_body,
        out_shape=jax.ShapeDtypeStruct((b, 2), _F32),
    )(*[a.astype(_F32) for a in args])
    return out


# speedup vs baseline: 2.2729x; 1.6027x over previous
import jax
import jax.numpy as jnp
from jax import lax
from jax.experimental import pallas as pl
from jax.experimental.pallas import tpu as pltpu

_ROI = 1500
_N = 1536
_S = 128
_NS = _N // _S
_KS = (1200, 900, 600, 300)

_DW, _DB, _UW, _UB, _PW, _PB = 0, 4, 8, 12, 16, 20
_SB, _BW, _BB, _EW0, _EW1, _EB = 24, 25, 26, 27, 28, 29

_F32 = jnp.float32
_BF16 = jnp.bfloat16
_CN = (((1,), (0,)), ((), ()))


def _unet_body(g_ref, sw_ref, sc_ref, out_ref,
               b0, b1, b2, b3, rank_ref, rs_ref, s_sc, k_sc, v_sc, mv_ref):
    bmats = (b0, b1, b2, b3, b0)

    def sc(i):
        return sc_ref[i]

    def t(x):
        return jnp.transpose(x)

    def mv_g(x):
        xr = t(x)

        def strip(i, c):
            left = g_ref[0, pl.ds(i * _S, _S), :]
            mv_ref[pl.ds(i * _S, _S), :] = jnp.sum(
                left * xr, axis=1, keepdims=True)
            return c

        lax.fori_loop(0, _NS, strip, 0)
        return mv_ref[...]

    def mv_b(bref, x):
        xr = t(x)

        def strip(i, cc):
            left = bref[pl.ds(i * _S, _S), :].astype(_F32)
            mv_ref[pl.ds(i * _S, _S), :] = jnp.sum(
                left * xr, axis=1, keepdims=True)
            return cc

        lax.fori_loop(0, _NS, strip, 0)
        return mv_ref[...]

    def sigmoid(x):
        e = jnp.exp(-jnp.abs(x))
        return jnp.where(x >= 0, 1.0 / (1.0 + e), e / (1.0 + e))

    def rank_topk(s, key, valid, k):
        srow, krow = t(s), t(key)
        vrow = t(valid) > 0.0
        s_sc[...] = s
        k_sc[...] = key

        def strip(i, carry):
            sc_ = s_sc[pl.ds(i * _S, _S), :]
            kc_ = k_sc[pl.ds(i * _S, _S), :]
            beats = ((srow > sc_) | ((srow == sc_) & (krow < kc_))) & vrow
            rank_ref[pl.ds(i * _S, _S), :] = jnp.sum(
                jnp.where(beats, 1.0, 0.0), axis=1, keepdims=True)
            return carry

        lax.fori_loop(0, _NS, strip, 0)
        rank = rank_ref[...]
        sel = jnp.where((valid > 0.0) & (rank < float(k)), 1.0, 0.0)
        return sel, rank

    def pool_graph(bsrc, bdst, sel):
        selrow = t(sel)
        v_sc[...] = sel
        bfull = bsrc[...]

        def strip(i, carry):
            left = bsrc[pl.ds(i * _S, _S), :]
            m = lax.dot_general(left, bfull, _CN,
                                preferred_element_type=_F32)
            selc = v_sc[pl.ds(i * _S, _S), :]
            bn = jnp.where((m > 0.0) & (selc > 0.0) & (selrow > 0.0),
                           1.0, 0.0)
            bdst[pl.ds(i * _S, _S), :] = bn.astype(_BF16)
            rs_ref[pl.ds(i * _S, _S), :] = jnp.sum(bn, axis=1, keepdims=True)
            return carry

        lax.fori_loop(0, _NS, strip, 0)
        return 1.0 / (rs_ref[...] + 1e-8)

    iota_col = lax.broadcasted_iota(jnp.int32, (_N, 1), 0)
    valid0 = jnp.where(iota_col < _ROI, 1.0, 0.0)

    def strip_b0(i, carry):
        b0[pl.ds(i * _S, _S), :] = (
            g_ref[0, pl.ds(i * _S, _S), :] != 0.0).astype(_BF16)
        return carry

    lax.fori_loop(0, _NS, strip_b0, 0)

    h0 = jnp.maximum(mv_g(sw_ref[...]) + sc(_SB), 0.0) * valid0
    org_h = h0

    masks = [valid0]
    invss = [None]
    downs = []
    h = h0
    key = iota_col.astype(_F32)
    for lvl in range(4):
        if lvl == 0:
            tv = mv_g(h)
        else:
            tv = mv_b(bmats[lvl], h) * invss[lvl]
        h = jnp.maximum(tv * sc(_DW + lvl) + sc(_DB + lvl), 0.0) * masks[lvl]
        downs.append(h)
        s = sigmoid(h * sc(_PW + lvl) + sc(_PB + lvl))
        sel, rank = rank_topk(s, key, masks[lvl], _KS[lvl])
        h = h * s * sel
        key = rank
        invss.append(pool_graph(bmats[lvl], bmats[lvl + 1], sel))
        masks.append(sel)

    tv = mv_b(bmats[4], h) * invss[4]
    h = jnp.maximum(tv * sc(_BW) + sc(_BB), 0.0) * masks[4]

    for i in range(4):
        up = 3 - i
        if up == 0:
            tv = mv_g(h)
        else:
            tv = mv_b(bmats[up], h) * invss[up]
        h = jnp.maximum(tv * sc(_UW + i) + sc(_UB + i), 0.0) * masks[up]
        h = h + downs[up]

    acc = mv_g(h) * sc(_EW0)
    acc = acc + mv_g(org_h) * sc(_EW1)
    res = jnp.maximum(acc + sc(_EB), 0.0) * valid0
    out_ref[0] = t(res)


def _mlp_body(x_ref, g1, b1, w1, c1, g2, b2, w2, c2,
              g3, b3, w3, c3, g4, b4, w4, c4, o_ref):
    def bn(x, ga, be):
        m = jnp.mean(x, axis=0, keepdims=True)
        v = jnp.mean((x - m) ** 2, axis=0, keepdims=True)
        return (x - m) / jnp.sqrt(v + 1e-5) * ga[...] + be[...]

    def dot(a, b):
        return lax.dot_general(a, b, _CN, precision=lax.Precision.HIGHEST,
                               preferred_element_type=_F32)

    h = x_ref[...]
    for ga, be, w, c in ((g1, b1, w1, c1), (g2, b2, w2, c2),
                         (g3, b3, w3, c3), (g4, b4, w4, c4)):
        h = jnp.maximum(bn(h, ga, be), 0.0)
        h = dot(h, w[...]) + c[...]
    z = h - jnp.max(h, axis=1, keepdims=True)
    e = jnp.exp(z)
    o_ref[...] = e / jnp.sum(e, axis=1, keepdims=True)


def kernel(g, params):
    b = g.shape[0]
    pad = _N - _ROI
    gp = jnp.pad(g.astype(_F32), ((0, 0), (0, pad), (0, pad)))
    swp = jnp.pad(params['start_w'].astype(_F32), ((0, pad), (0, 0)))

    vals = []
    for key in ('down_w', 'down_b', 'up_w', 'up_b', 'pool_w', 'pool_b'):
        vals += [params[key][i].reshape(()) for i in range(4)]
    vals += [params['start_b'].reshape(()),
             params['bottom_w'].reshape(()), params['bottom_b'].reshape(()),
             params['end_w'][0, 0], params['end_w'][1, 0],
             params['end_b'].reshape(())]
    scal = jnp.stack([v.astype(_F32) for v in vals])

    unet_out = pl.pallas_call(
        _unet_body,
        grid=(b,),
        in_specs=[
            pl.BlockSpec((1, _N, _N), lambda i: (i, 0, 0),
                         pipeline_mode=pl.Buffered(buffer_count=1)),
            pl.BlockSpec((_N, 1), lambda i: (0, 0)),
            pl.BlockSpec(memory_space=pltpu.SMEM),
        ],
        out_specs=pl.BlockSpec((1, 1, _N), lambda i: (i, 0, 0)),
        out_shape=jax.ShapeDtypeStruct((b, 1, _N), _F32),
        scratch_shapes=[pltpu.VMEM((_N, _N), _BF16)] * 4
        + [pltpu.VMEM((_N, 1), _F32)] * 6,
    )(gp, swp, scal)
    unet_out = unet_out.reshape(b, _N)

    w1p = jnp.pad(params['fl1_w'].astype(_F32), ((0, pad), (0, 0)))
    g1p = jnp.pad(params['bn1_g'].astype(_F32), (0, pad)).reshape(1, _N)
    b1p = jnp.pad(params['bn1_b'].astype(_F32), (0, pad)).reshape(1, _N)

    args = [unet_out,
            g1p, b1p, w1p, params['fl1_b'].reshape(1, -1),
            params['bn2_g'].reshape(1, -1), params['bn2_b'].reshape(1, -1),
            params['fl2_w'], params['fl2_b'].reshape(1, -1),
            params['bn3_g'].reshape(1, -1), params['bn3_b'].reshape(1, -1),
            params['fl3_w'], params['fl3_b'].reshape(1, -1),
            params['bn4_g'].reshape(1, -1), params['bn4_b'].reshape(1, -1),
            params['fl4_w'], params['fl4_b'].reshape(1, -1)]

    out = pl.pallas_call(
        _mlp_body,
        out_shape=jax.ShapeDtypeStruct((b, 2), _F32),
    )(*[a.astype(_F32) for a in args])
    return out


# speedup vs baseline: 3.1033x; 1.3653x over previous
import jax
import jax.numpy as jnp
from jax import lax
from jax.experimental import pallas as pl
from jax.experimental.pallas import tpu as pltpu

_ROI = 1500
_N = 1536
_S = 128
_NS = _N // _S
_KS = (1200, 900, 600, 300)

_DW, _DB, _UW, _UB, _PW, _PB = 0, 4, 8, 12, 16, 20
_SB, _BW, _BB, _EW0, _EW1, _EB = 24, 25, 26, 27, 28, 29

_F32 = jnp.float32
_BF16 = jnp.bfloat16
_F8 = jnp.float8_e4m3fn
_CN = (((1,), (0,)), ((), ()))


def _unet_body(g_ref, sw_ref, sc_ref, out_ref,
               gp_ref, b0, b1, b2, b3, rank_ref, rs_ref, s_sc, k_sc, v_sc,
               mv_ref):
    bmats = (b0, b1, b2, b3, b0)

    def sc(i):
        return sc_ref[i]

    def t(x):
        return jnp.transpose(x)

    def strip_gp(i, c):
        gp_ref[pl.ds(i * _S, _S), 0:_ROI] = g_ref[0, pl.ds(i * _S, _S), :]
        gp_ref[pl.ds(i * _S, _S), _ROI:_N] = jnp.zeros((_S, _N - _ROI), _F32)
        return c

    lax.fori_loop(0, _NS - 1, strip_gp, 0)
    gp_ref[pl.ds(11 * _S, _S), :] = jnp.zeros((_S, _N), _F32)
    gp_ref[pl.ds(11 * _S, _ROI - 11 * _S), 0:_ROI] = (
        g_ref[0, pl.ds(11 * _S, _ROI - 11 * _S), :])

    def mv_g(x):
        xr = t(x)

        def strip(i, c):
            left = gp_ref[pl.ds(i * _S, _S), :]
            mv_ref[pl.ds(i * _S, _S), :] = jnp.sum(
                left * xr, axis=1, keepdims=True)
            return c

        lax.fori_loop(0, _NS, strip, 0)
        return mv_ref[...]

    def mv_b(bref, x):
        xr = t(x)

        def strip(i, cc):
            left = bref[pl.ds(i * _S, _S), :].astype(_F32)
            mv_ref[pl.ds(i * _S, _S), :] = jnp.sum(
                left * xr, axis=1, keepdims=True)
            return cc

        lax.fori_loop(0, _NS, strip, 0)
        return mv_ref[...]

    def sigmoid(x):
        e = jnp.exp(-jnp.abs(x))
        return jnp.where(x >= 0, 1.0 / (1.0 + e), e / (1.0 + e))

    def rank_topk(s, key, valid, k):
        sm = jnp.where(valid > 0.0, s, -1.0)
        srow, krow = t(sm), t(key)
        s_sc[...] = sm
        k_sc[...] = key

        def strip(i, carry):
            sc_ = s_sc[pl.ds(i * _S, _S), :]
            kc_ = k_sc[pl.ds(i * _S, _S), :]
            beats = (srow > sc_) | ((srow == sc_) & (krow < kc_))
            rank_ref[pl.ds(i * _S, _S), :] = jnp.sum(
                jnp.where(beats, 1.0, 0.0), axis=1, keepdims=True)
            return carry

        lax.fori_loop(0, _NS, strip, 0)
        rank = rank_ref[...]
        sel = jnp.where((valid > 0.0) & (rank < float(k)), 1.0, 0.0)
        return sel, rank

    def pool_graph(bsrc, bdst, sel):
        selrow = t(sel)
        v_sc[...] = sel
        bfull = bsrc[...]

        def strip(i, carry):
            left = bsrc[pl.ds(i * _S, _S), :]
            m = lax.dot_general(left, bfull, _CN,
                                preferred_element_type=_F32)
            selc = v_sc[pl.ds(i * _S, _S), :]
            bn = jnp.where((m > 0.0) & (selc > 0.0) & (selrow > 0.0),
                           1.0, 0.0)
            bdst[pl.ds(i * _S, _S), :] = bn.astype(_F8)
            rs_ref[pl.ds(i * _S, _S), :] = jnp.sum(bn, axis=1, keepdims=True)
            return carry

        lax.fori_loop(0, _NS, strip, 0)
        return 1.0 / (rs_ref[...] + 1e-8)

    iota_col = lax.broadcasted_iota(jnp.int32, (_N, 1), 0)
    valid0 = jnp.where(iota_col < _ROI, 1.0, 0.0)

    def strip_b0(i, carry):
        b0[pl.ds(i * _S, _S), :] = (
            gp_ref[pl.ds(i * _S, _S), :] != 0.0).astype(_F8)
        return carry

    lax.fori_loop(0, _NS, strip_b0, 0)

    h0 = jnp.maximum(mv_g(sw_ref[...]) + sc(_SB), 0.0) * valid0
    org_h = h0

    masks = [valid0]
    invss = [None]
    downs = []
    h = h0
    key = iota_col.astype(_F32)
    for lvl in range(4):
        if lvl == 0:
            tv = mv_g(h)
        else:
            tv = mv_b(bmats[lvl], h) * invss[lvl]
        h = jnp.maximum(tv * sc(_DW + lvl) + sc(_DB + lvl), 0.0) * masks[lvl]
        downs.append(h)
        s = sigmoid(h * sc(_PW + lvl) + sc(_PB + lvl))
        sel, rank = rank_topk(s, key, masks[lvl], _KS[lvl])
        h = h * s * sel
        key = rank
        invss.append(pool_graph(bmats[lvl], bmats[lvl + 1], sel))
        masks.append(sel)

    tv = mv_b(bmats[4], h) * invss[4]
    h = jnp.maximum(tv * sc(_BW) + sc(_BB), 0.0) * masks[4]

    for i in range(4):
        up = 3 - i
        if up == 0:
            tv = mv_g(h)
        else:
            tv = mv_b(bmats[up], h) * invss[up]
        h = jnp.maximum(tv * sc(_UW + i) + sc(_UB + i), 0.0) * masks[up]
        h = h + downs[up]

    acc = mv_g(h) * sc(_EW0)
    acc = acc + mv_g(org_h) * sc(_EW1)
    res = jnp.maximum(acc + sc(_EB), 0.0) * valid0
    out_ref[0] = t(res)


def _mlp_body(x_ref, g1, b1, w1, c1, g2, b2, w2, c2,
              g3, b3, w3, c3, g4, b4, w4, c4, o_ref):
    def bn(x, ga, be):
        m = jnp.mean(x, axis=0, keepdims=True)
        v = jnp.mean((x - m) ** 2, axis=0, keepdims=True)
        return (x - m) / jnp.sqrt(v + 1e-5) * ga[...] + be[...]

    def dot(a, b):
        return lax.dot_general(a, b, _CN, precision=lax.Precision.HIGHEST,
                               preferred_element_type=_F32)

    h = x_ref[...]
    for ga, be, w, c in ((g1, b1, w1, c1), (g2, b2, w2, c2),
                         (g3, b3, w3, c3), (g4, b4, w4, c4)):
        h = jnp.maximum(bn(h, ga, be), 0.0)
        h = dot(h, w[...]) + c[...]
    z = h - jnp.max(h, axis=1, keepdims=True)
    e = jnp.exp(z)
    o_ref[...] = e / jnp.sum(e, axis=1, keepdims=True)


def kernel(g, params):
    b = g.shape[0]
    pad = _N - _ROI
    swp = jnp.pad(params['start_w'].astype(_F32), ((0, pad), (0, 0)))

    vals = []
    for key in ('down_w', 'down_b', 'up_w', 'up_b', 'pool_w', 'pool_b'):
        vals += [params[key][i].reshape(()) for i in range(4)]
    vals += [params['start_b'].reshape(()),
             params['bottom_w'].reshape(()), params['bottom_b'].reshape(()),
             params['end_w'][0, 0], params['end_w'][1, 0],
             params['end_b'].reshape(())]
    scal = jnp.stack([v.astype(_F32) for v in vals])

    unet_out = pl.pallas_call(
        _unet_body,
        grid=(b,),
        in_specs=[
            pl.BlockSpec((1, _ROI, _ROI), lambda i: (i, 0, 0),
                         pipeline_mode=pl.Buffered(buffer_count=2)),
            pl.BlockSpec((_N, 1), lambda i: (0, 0)),
            pl.BlockSpec(memory_space=pltpu.SMEM),
        ],
        out_specs=pl.BlockSpec((1, 1, _N), lambda i: (i, 0, 0)),
        out_shape=jax.ShapeDtypeStruct((b, 1, _N), _F32),
        scratch_shapes=[pltpu.VMEM((_N, _N), _F32)]
        + [pltpu.VMEM((_N, _N), _F8)] * 4
        + [pltpu.VMEM((_N, 1), _F32)] * 6,
    )(g.astype(_F32), swp, scal)
    unet_out = unet_out.reshape(b, _N)

    w1p = jnp.pad(params['fl1_w'].astype(_F32), ((0, pad), (0, 0)))
    g1p = jnp.pad(params['bn1_g'].astype(_F32), (0, pad)).reshape(1, _N)
    b1p = jnp.pad(params['bn1_b'].astype(_F32), (0, pad)).reshape(1, _N)

    args = [unet_out,
            g1p, b1p, w1p, params['fl1_b'].reshape(1, -1),
            params['bn2_g'].reshape(1, -1), params['bn2_b'].reshape(1, -1),
            params['fl2_w'], params['fl2_b'].reshape(1, -1),
            params['bn3_g'].reshape(1, -1), params['bn3_b'].reshape(1, -1),
            params['fl3_w'], params['fl3_b'].reshape(1, -1),
            params['bn4_g'].reshape(1, -1), params['bn4_b'].reshape(1, -1),
            params['fl4_w'], params['fl4_b'].reshape(1, -1)]

    out = pl.pallas_call(
        _mlp_body,
        out_shape=jax.ShapeDtypeStruct((b, 2), _F32),
    )(*[a.astype(_F32) for a in args])
    return out


# speedup vs baseline: 3.8393x; 1.2372x over previous
import jax
import jax.numpy as jnp
from jax import lax
from jax.experimental import pallas as pl
from jax.experimental.pallas import tpu as pltpu

_ROI = 1500
_N = 1536
_S = 128
_NS = _N // _S
_KS = (1200, 900, 600, 300)

_DW, _DB, _UW, _UB, _PW, _PB = 0, 4, 8, 12, 16, 20
_SB, _BW, _BB, _EW0, _EW1, _EB = 24, 25, 26, 27, 28, 29

_F32 = jnp.float32
_BF16 = jnp.bfloat16
_F8 = jnp.float8_e4m3fn
_CN = (((1,), (0,)), ((), ()))


def _unet_body(g_ref, sw_ref, sc_ref, out_ref,
               gp_ref, b0, b1, b2, b3, rank_ref, rs_ref, s_sc, k_sc, v_sc,
               mv_ref):
    bmats = (b0, b1, b2, b3, b0)

    def sc(i):
        return sc_ref[i]

    def t(x):
        return jnp.transpose(x)

    def strip_gp(i, c):
        gp_ref[pl.ds(i * _S, _S), 0:_ROI] = g_ref[0, pl.ds(i * _S, _S), :]
        gp_ref[pl.ds(i * _S, _S), _ROI:_N] = jnp.zeros((_S, _N - _ROI), _F32)
        b0[pl.ds(i * _S, _S), :] = (
            gp_ref[pl.ds(i * _S, _S), :] != 0.0).astype(_F8)
        return c

    lax.fori_loop(0, _NS - 1, strip_gp, 0)
    gp_ref[pl.ds(11 * _S, _S), :] = jnp.zeros((_S, _N), _F32)
    gp_ref[pl.ds(11 * _S, _ROI - 11 * _S), 0:_ROI] = (
        g_ref[0, pl.ds(11 * _S, _ROI - 11 * _S), :])
    b0[pl.ds(11 * _S, _S), :] = (
        gp_ref[pl.ds(11 * _S, _S), :] != 0.0).astype(_F8)

    def mv_g(x):
        xr = t(x)

        def strip(i, c):
            left = gp_ref[pl.ds(i * _S, _S), :]
            mv_ref[pl.ds(i * _S, _S), :] = jnp.sum(
                left * xr, axis=1, keepdims=True)
            return c

        lax.fori_loop(0, _NS, strip, 0, unroll=2)
        return mv_ref[...]

    def mv_b(bref, x):
        xr = t(x)

        def strip(i, cc):
            left = bref[pl.ds(i * _S, _S), :].astype(_F32)
            mv_ref[pl.ds(i * _S, _S), :] = jnp.sum(
                left * xr, axis=1, keepdims=True)
            return cc

        lax.fori_loop(0, _NS, strip, 0, unroll=2)
        return mv_ref[...]

    def sigmoid(x):
        e = jnp.exp(-jnp.abs(x))
        return jnp.where(x >= 0, 1.0 / (1.0 + e), e / (1.0 + e))

    def rank_topk(s, key, valid, k):
        sm = jnp.where(valid > 0.0, s, -1.0)
        srow, krow = t(sm), t(key)
        s_sc[...] = sm
        k_sc[...] = key

        def strip(i, carry):
            sc_ = s_sc[pl.ds(i * _S, _S), :]
            kc_ = k_sc[pl.ds(i * _S, _S), :]
            beats = (srow > sc_) | ((srow == sc_) & (krow < kc_))
            rank_ref[pl.ds(i * _S, _S), :] = jnp.sum(
                jnp.where(beats, 1.0, 0.0), axis=1, keepdims=True)
            return carry

        lax.fori_loop(0, _NS, strip, 0, unroll=2)
        rank = rank_ref[...]
        sel = jnp.where((valid > 0.0) & (rank < float(k)), 1.0, 0.0)
        return sel, rank

    def pool_graph(bsrc, bdst, sel):
        selrow = t(sel)
        v_sc[...] = sel
        bfull = bsrc[...]

        def strip(i, carry):
            left = bsrc[pl.ds(i * _S, _S), :]
            m = lax.dot_general(left, bfull, _CN,
                                preferred_element_type=_F32)
            selc = v_sc[pl.ds(i * _S, _S), :]
            bn = jnp.where((m > 0.0) & (selc > 0.0) & (selrow > 0.0),
                           1.0, 0.0)
            bdst[pl.ds(i * _S, _S), :] = bn.astype(_F8)
            rs_ref[pl.ds(i * _S, _S), :] = jnp.sum(bn, axis=1, keepdims=True)
            return carry

        lax.fori_loop(0, _NS, strip, 0, unroll=2)
        return 1.0 / (rs_ref[...] + 1e-8)

    iota_col = lax.broadcasted_iota(jnp.int32, (_N, 1), 0)
    valid0 = jnp.where(iota_col < _ROI, 1.0, 0.0)

    h0 = jnp.maximum(mv_g(sw_ref[...]) + sc(_SB), 0.0) * valid0
    org_h = h0

    masks = [valid0]
    invss = [None]
    downs = []
    h = h0
    key = iota_col.astype(_F32)
    for lvl in range(4):
        if lvl == 0:
            tv = mv_g(h)
        else:
            tv = mv_b(bmats[lvl], h) * invss[lvl]
        h = jnp.maximum(tv * sc(_DW + lvl) + sc(_DB + lvl), 0.0) * masks[lvl]
        downs.append(h)
        s = sigmoid(h * sc(_PW + lvl) + sc(_PB + lvl))
        sel, rank = rank_topk(s, key, masks[lvl], _KS[lvl])
        h = h * s * sel
        key = rank
        invss.append(pool_graph(bmats[lvl], bmats[lvl + 1], sel))
        masks.append(sel)

    tv = mv_b(bmats[4], h) * invss[4]
    h = jnp.maximum(tv * sc(_BW) + sc(_BB), 0.0) * masks[4]

    for i in range(4):
        up = 3 - i
        if up == 0:
            tv = mv_g(h)
        else:
            tv = mv_b(bmats[up], h) * invss[up]
        h = jnp.maximum(tv * sc(_UW + i) + sc(_UB + i), 0.0) * masks[up]
        h = h + downs[up]

    acc = mv_g(h * sc(_EW0) + org_h * sc(_EW1))
    res = jnp.maximum(acc + sc(_EB), 0.0) * valid0
    out_ref[0] = t(res)


def _mlp_body(x_ref, g1, b1, w1, c1, g2, b2, w2, c2,
              g3, b3, w3, c3, g4, b4, w4, c4, o_ref):
    def bn(x, ga, be):
        m = jnp.mean(x, axis=0, keepdims=True)
        v = jnp.mean((x - m) ** 2, axis=0, keepdims=True)
        return (x - m) / jnp.sqrt(v + 1e-5) * ga[...] + be[...]

    def dot(a, b):
        return lax.dot_general(a, b, _CN, precision=lax.Precision.HIGHEST,
                               preferred_element_type=_F32)

    h = x_ref[...]
    for ga, be, w, c in ((g1, b1, w1, c1), (g2, b2, w2, c2),
                         (g3, b3, w3, c3), (g4, b4, w4, c4)):
        h = jnp.maximum(bn(h, ga, be), 0.0)
        h = dot(h, w[...]) + c[...]
    z = h - jnp.max(h, axis=1, keepdims=True)
    e = jnp.exp(z)
    o_ref[...] = e / jnp.sum(e, axis=1, keepdims=True)


def kernel(g, params):
    b = g.shape[0]
    pad = _N - _ROI
    swp = jnp.pad(params['start_w'].astype(_F32), ((0, pad), (0, 0)))

    vals = []
    for key in ('down_w', 'down_b', 'up_w', 'up_b', 'pool_w', 'pool_b'):
        vals += [params[key][i].reshape(()) for i in range(4)]
    vals += [params['start_b'].reshape(()),
             params['bottom_w'].reshape(()), params['bottom_b'].reshape(()),
             params['end_w'][0, 0], params['end_w'][1, 0],
             params['end_b'].reshape(())]
    scal = jnp.stack([v.astype(_F32) for v in vals])

    unet_out = pl.pallas_call(
        _unet_body,
        grid=(b,),
        in_specs=[
            pl.BlockSpec((1, _ROI, _ROI), lambda i: (i, 0, 0),
                         pipeline_mode=pl.Buffered(buffer_count=2)),
            pl.BlockSpec((_N, 1), lambda i: (0, 0)),
            pl.BlockSpec(memory_space=pltpu.SMEM),
        ],
        out_specs=pl.BlockSpec((1, 1, _N), lambda i: (i, 0, 0)),
        out_shape=jax.ShapeDtypeStruct((b, 1, _N), _F32),
        scratch_shapes=[pltpu.VMEM((_N, _N), _F32)]
        + [pltpu.VMEM((_N, _N), _F8)] * 4
        + [pltpu.VMEM((_N, 1), _F32)] * 6,
    )(g.astype(_F32), swp, scal)
    unet_out = unet_out.reshape(b, _N)

    w1p = jnp.pad(params['fl1_w'].astype(_F32), ((0, pad), (0, 0)))
    g1p = jnp.pad(params['bn1_g'].astype(_F32), (0, pad)).reshape(1, _N)
    b1p = jnp.pad(params['bn1_b'].astype(_F32), (0, pad)).reshape(1, _N)

    args = [unet_out,
            g1p, b1p, w1p, params['fl1_b'].reshape(1, -1),
            params['bn2_g'].reshape(1, -1), params['bn2_b'].reshape(1, -1),
            params['fl2_w'], params['fl2_b'].reshape(1, -1),
            params['bn3_g'].reshape(1, -1), params['bn3_b'].reshape(1, -1),
            params['fl3_w'], params['fl3_b'].reshape(1, -1),
            params['bn4_g'].reshape(1, -1), params['bn4_b'].reshape(1, -1),
            params['fl4_w'], params['fl4_b'].reshape(1, -1)]

    out = pl.pallas_call(
        _mlp_body,
        out_shape=jax.ShapeDtypeStruct((b, 2), _F32),
    )(*[a.astype(_F32) for a in args])
    return out


# speedup vs baseline: 4.0282x; 1.0492x over previous
import jax
import jax.numpy as jnp
from jax import lax
from jax.experimental import pallas as pl
from jax.experimental.pallas import tpu as pltpu

_ROI = 1500
_N = 1536
_S = 128
_NS = _N // _S
_KS = (1200, 900, 600, 300)

_DW, _DB, _UW, _UB, _PW, _PB = 0, 4, 8, 12, 16, 20
_SB, _BW, _BB, _EW0, _EW1, _EB = 24, 25, 26, 27, 28, 29

_F32 = jnp.float32
_BF16 = jnp.bfloat16
_F8 = jnp.float8_e4m3fn
_CN = (((1,), (0,)), ((), ()))


def _unet_body(g_ref, sw_ref, sc_ref, out_ref,
               gp_ref, b0, b1, b2, b3, rank_ref, rs_ref, s_sc, k_sc, v_sc,
               mv_ref):
    bmats = (b0, b1, b2, b3, b0)

    def sc(i):
        return sc_ref[i]

    def t(x):
        return jnp.transpose(x)

    def strip_gp(i, c):
        gp_ref[pl.ds(i * _S, _S), 0:_ROI] = g_ref[0, pl.ds(i * _S, _S), :]
        gp_ref[pl.ds(i * _S, _S), _ROI:_N] = jnp.zeros((_S, _N - _ROI), _F32)
        b0[pl.ds(i * _S, _S), :] = (
            gp_ref[pl.ds(i * _S, _S), :] != 0.0).astype(_F8)
        return c

    lax.fori_loop(0, _NS - 1, strip_gp, 0)
    gp_ref[pl.ds(11 * _S, _S), :] = jnp.zeros((_S, _N), _F32)
    gp_ref[pl.ds(11 * _S, _ROI - 11 * _S), 0:_ROI] = (
        g_ref[0, pl.ds(11 * _S, _ROI - 11 * _S), :])
    b0[pl.ds(11 * _S, _S), :] = (
        gp_ref[pl.ds(11 * _S, _S), :] != 0.0).astype(_F8)

    def mv_g(x):
        xr = t(x)

        def strip(i, c):
            left = gp_ref[pl.ds(i * _S, _S), :]
            mv_ref[pl.ds(i * _S, _S), :] = jnp.sum(
                left * xr, axis=1, keepdims=True)
            return c

        lax.fori_loop(0, _NS, strip, 0, unroll=3)
        return mv_ref[...]

    def mv_b(bref, x):
        xr = t(x)

        def strip(i, cc):
            left = bref[pl.ds(i * _S, _S), :].astype(_F32)
            mv_ref[pl.ds(i * _S, _S), :] = jnp.sum(
                left * xr, axis=1, keepdims=True)
            return cc

        lax.fori_loop(0, _NS, strip, 0, unroll=3)
        return mv_ref[...]

    def sigmoid(x):
        e = jnp.exp(-jnp.abs(x))
        return jnp.where(x >= 0, 1.0 / (1.0 + e), e / (1.0 + e))

    def rank_topk(s, key, valid, k):
        sm = jnp.where(valid > 0.0, s, -1.0)
        srow, krow = t(sm), t(key)
        s_sc[...] = sm
        k_sc[...] = key

        def strip(i, carry):
            sc_ = s_sc[pl.ds(i * _S, _S), :]
            kc_ = k_sc[pl.ds(i * _S, _S), :]
            beats = (srow > sc_) | ((srow == sc_) & (krow < kc_))
            rank_ref[pl.ds(i * _S, _S), :] = jnp.sum(
                jnp.where(beats, 1.0, 0.0), axis=1, keepdims=True)
            return carry

        lax.fori_loop(0, _NS, strip, 0, unroll=3)
        rank = rank_ref[...]
        sel = jnp.where((valid > 0.0) & (rank < float(k)), 1.0, 0.0)
        return sel, rank

    def pool_graph(bsrc, bdst, sel):
        selrow = t(sel)
        v_sc[...] = sel
        bfull = bsrc[...]

        def strip(i, carry):
            left = bsrc[pl.ds(i * _S, _S), :]
            m = lax.dot_general(left, bfull, _CN,
                                preferred_element_type=_F32)
            selc = v_sc[pl.ds(i * _S, _S), :]
            bn = jnp.where((m > 0.0) & (selc > 0.0) & (selrow > 0.0),
                           1.0, 0.0)
            bdst[pl.ds(i * _S, _S), :] = bn.astype(_F8)
            rs_ref[pl.ds(i * _S, _S), :] = jnp.sum(bn, axis=1, keepdims=True)
            return carry

        lax.fori_loop(0, _NS, strip, 0, unroll=3)
        return 1.0 / (rs_ref[...] + 1e-8)

    iota_col = lax.broadcasted_iota(jnp.int32, (_N, 1), 0)
    valid0 = jnp.where(iota_col < _ROI, 1.0, 0.0)

    h0 = jnp.maximum(mv_g(sw_ref[...]) + sc(_SB), 0.0) * valid0
    org_h = h0

    masks = [valid0]
    invss = [None]
    downs = []
    h = h0
    key = iota_col.astype(_F32)
    for lvl in range(4):
        if lvl == 0:
            tv = mv_g(h)
        else:
            tv = mv_b(bmats[lvl], h) * invss[lvl]
        h = jnp.maximum(tv * sc(_DW + lvl) + sc(_DB + lvl), 0.0) * masks[lvl]
        downs.append(h)
        s = sigmoid(h * sc(_PW + lvl) + sc(_PB + lvl))
        sel, rank = rank_topk(s, key, masks[lvl], _KS[lvl])
        h = h * s * sel
        key = rank
        invss.append(pool_graph(bmats[lvl], bmats[lvl + 1], sel))
        masks.append(sel)

    tv = mv_b(bmats[4], h) * invss[4]
    h = jnp.maximum(tv * sc(_BW) + sc(_BB), 0.0) * masks[4]

    for i in range(4):
        up = 3 - i
        if up == 0:
            tv = mv_g(h)
        else:
            tv = mv_b(bmats[up], h) * invss[up]
        h = jnp.maximum(tv * sc(_UW + i) + sc(_UB + i), 0.0) * masks[up]
        h = h + downs[up]

    acc = mv_g(h * sc(_EW0) + org_h * sc(_EW1))
    res = jnp.maximum(acc + sc(_EB), 0.0) * valid0
    out_ref[0] = t(res)


def _mlp_body(x_ref, g1, b1, w1, c1, g2, b2, w2, c2,
              g3, b3, w3, c3, g4, b4, w4, c4, o_ref):
    def bn(x, ga, be):
        m = jnp.mean(x, axis=0, keepdims=True)
        v = jnp.mean((x - m) ** 2, axis=0, keepdims=True)
        return (x - m) / jnp.sqrt(v + 1e-5) * ga[...] + be[...]

    def dot(a, b):
        return lax.dot_general(a, b, _CN, precision=lax.Precision.HIGHEST,
                               preferred_element_type=_F32)

    h = x_ref[...]
    for ga, be, w, c in ((g1, b1, w1, c1), (g2, b2, w2, c2),
                         (g3, b3, w3, c3), (g4, b4, w4, c4)):
        h = jnp.maximum(bn(h, ga, be), 0.0)
        h = dot(h, w[...]) + c[...]
    z = h - jnp.max(h, axis=1, keepdims=True)
    e = jnp.exp(z)
    o_ref[...] = e / jnp.sum(e, axis=1, keepdims=True)


def kernel(g, params):
    b = g.shape[0]
    pad = _N - _ROI
    swp = jnp.pad(params['start_w'].astype(_F32), ((0, pad), (0, 0)))

    vals = []
    for key in ('down_w', 'down_b', 'up_w', 'up_b', 'pool_w', 'pool_b'):
        vals += [params[key][i].reshape(()) for i in range(4)]
    vals += [params['start_b'].reshape(()),
             params['bottom_w'].reshape(()), params['bottom_b'].reshape(()),
             params['end_w'][0, 0], params['end_w'][1, 0],
             params['end_b'].reshape(())]
    scal = jnp.stack([v.astype(_F32) for v in vals])

    unet_out = pl.pallas_call(
        _unet_body,
        grid=(b,),
        in_specs=[
            pl.BlockSpec((1, _ROI, _ROI), lambda i: (i, 0, 0),
                         pipeline_mode=pl.Buffered(buffer_count=2)),
            pl.BlockSpec((_N, 1), lambda i: (0, 0)),
            pl.BlockSpec(memory_space=pltpu.SMEM),
        ],
        out_specs=pl.BlockSpec((1, 1, _N), lambda i: (i, 0, 0)),
        out_shape=jax.ShapeDtypeStruct((b, 1, _N), _F32),
        scratch_shapes=[pltpu.VMEM((_N, _N), _F32)]
        + [pltpu.VMEM((_N, _N), _F8)] * 4
        + [pltpu.VMEM((_N, 1), _F32)] * 6,
    )(g.astype(_F32), swp, scal)
    unet_out = unet_out.reshape(b, _N)

    w1p = jnp.pad(params['fl1_w'].astype(_F32), ((0, pad), (0, 0)))
    g1p = jnp.pad(params['bn1_g'].astype(_F32), (0, pad)).reshape(1, _N)
    b1p = jnp.pad(params['bn1_b'].astype(_F32), (0, pad)).reshape(1, _N)

    args = [unet_out,
            g1p, b1p, w1p, params['fl1_b'].reshape(1, -1),
            params['bn2_g'].reshape(1, -1), params['bn2_b'].reshape(1, -1),
            params['fl2_w'], params['fl2_b'].reshape(1, -1),
            params['bn3_g'].reshape(1, -1), params['bn3_b'].reshape(1, -1),
            params['fl3_w'], params['fl3_b'].reshape(1, -1),
            params['bn4_g'].reshape(1, -1), params['bn4_b'].reshape(1, -1),
            params['fl4_w'], params['fl4_b'].reshape(1, -1)]

    out = pl.pallas_call(
        _mlp_body,
        out_shape=jax.ShapeDtypeStruct((b, 2), _F32),
    )(*[a.astype(_F32) for a in args])
    return out


# speedup vs baseline: 4.2828x; 1.0632x over previous
import jax
import jax.numpy as jnp
from jax import lax
from jax.experimental import pallas as pl
from jax.experimental.pallas import tpu as pltpu

_ROI = 1500
_N = 1536
_S = 128
_NS = _N // _S
_KS = (1200, 900, 600, 300)

_DW, _DB, _UW, _UB, _PW, _PB = 0, 4, 8, 12, 16, 20
_SB, _BW, _BB, _EW0, _EW1, _EB = 24, 25, 26, 27, 28, 29

_F32 = jnp.float32
_BF16 = jnp.bfloat16
_F8 = jnp.float8_e4m3fn
_CN = (((1,), (0,)), ((), ()))


def _unet_body(g_ref, sw_ref, sc_ref, out_ref,
               gp_ref, b0, b1, b2, b3, rank_ref, rs_ref, s_sc, k_sc, v_sc,
               mv_ref):
    bmats = (b0, b1, b2, b3, b0)

    def sc(i):
        return sc_ref[i]

    def t(x):
        return jnp.transpose(x)

    swrow = t(sw_ref[...])

    def strip_gp(i, c):
        gp_ref[pl.ds(i * _S, _S), 0:_ROI] = g_ref[0, pl.ds(i * _S, _S), :]
        gp_ref[pl.ds(i * _S, _S), _ROI:_N] = jnp.zeros((_S, _N - _ROI), _F32)
        gs = gp_ref[pl.ds(i * _S, _S), :]
        b0[pl.ds(i * _S, _S), :] = (gs != 0.0).astype(_F8)
        mv_ref[pl.ds(i * _S, _S), :] = jnp.sum(
            gs * swrow, axis=1, keepdims=True)
        return c

    lax.fori_loop(0, _NS - 1, strip_gp, 0)
    gp_ref[pl.ds(11 * _S, _S), :] = jnp.zeros((_S, _N), _F32)
    gp_ref[pl.ds(11 * _S, _ROI - 11 * _S), 0:_ROI] = (
        g_ref[0, pl.ds(11 * _S, _ROI - 11 * _S), :])
    gs_t = gp_ref[pl.ds(11 * _S, _S), :]
    b0[pl.ds(11 * _S, _S), :] = (gs_t != 0.0).astype(_F8)
    mv_ref[pl.ds(11 * _S, _S), :] = jnp.sum(
        gs_t * swrow, axis=1, keepdims=True)

    def mv_g(x):
        xr = t(x)

        def strip(i, c):
            left = gp_ref[pl.ds(i * _S, _S), :]
            mv_ref[pl.ds(i * _S, _S), :] = jnp.sum(
                left * xr, axis=1, keepdims=True)
            return c

        lax.fori_loop(0, _NS, strip, 0, unroll=3)
        return mv_ref[...]

    def mv_b(bref, x):
        xr = t(x)

        def strip(i, cc):
            left = bref[pl.ds(i * _S, _S), :].astype(_F32)
            mv_ref[pl.ds(i * _S, _S), :] = jnp.sum(
                left * xr, axis=1, keepdims=True)
            return cc

        lax.fori_loop(0, _NS, strip, 0, unroll=3)
        return mv_ref[...]

    def sigmoid(x):
        e = jnp.exp(-jnp.abs(x))
        return jnp.where(x >= 0, 1.0 / (1.0 + e), e / (1.0 + e))

    def rank_topk(s, key, valid, k):
        sm = jnp.where(valid > 0.0, s, -1.0)
        srow, krow = t(sm), t(key)
        s_sc[...] = sm
        k_sc[...] = key

        def strip(i, carry):
            sc_ = s_sc[pl.ds(i * _S, _S), :]
            kc_ = k_sc[pl.ds(i * _S, _S), :]
            beats = (srow > sc_) | ((srow == sc_) & (krow < kc_))
            rank_ref[pl.ds(i * _S, _S), :] = jnp.sum(
                jnp.where(beats, 1.0, 0.0), axis=1, keepdims=True)
            return carry

        lax.fori_loop(0, _NS, strip, 0, unroll=3)
        rank = rank_ref[...]
        sel = jnp.where((valid > 0.0) & (rank < float(k)), 1.0, 0.0)
        return sel, rank

    def pool_graph(bsrc, bdst, sel, hvec):
        selrow = t(sel)
        v_sc[...] = sel
        hrow = t(hvec)
        bfull = bsrc[...]

        def strip(i, carry):
            left = bsrc[pl.ds(i * _S, _S), :]
            m = lax.dot_general(left, bfull, _CN,
                                preferred_element_type=_F32)
            selc = v_sc[pl.ds(i * _S, _S), :]
            bn = jnp.where((m > 0.0) & (selc > 0.0) & (selrow > 0.0),
                           1.0, 0.0)
            bdst[pl.ds(i * _S, _S), :] = bn.astype(_F8)
            rs_ref[pl.ds(i * _S, _S), :] = jnp.sum(bn, axis=1, keepdims=True)
            mv_ref[pl.ds(i * _S, _S), :] = jnp.sum(
                bn * hrow, axis=1, keepdims=True)
            return carry

        lax.fori_loop(0, _NS, strip, 0, unroll=3)
        return 1.0 / (rs_ref[...] + 1e-8), mv_ref[...]

    iota_col = lax.broadcasted_iota(jnp.int32, (_N, 1), 0)
    valid0 = jnp.where(iota_col < _ROI, 1.0, 0.0)

    h0 = jnp.maximum(mv_ref[...] + sc(_SB), 0.0) * valid0
    org_h = h0

    masks = [valid0]
    invss = [None]
    downs = []
    h = h0
    key = iota_col.astype(_F32)
    tvp = None
    for lvl in range(4):
        if lvl == 0:
            tv = mv_g(h)
        else:
            tv = tvp * invss[lvl]
        h = jnp.maximum(tv * sc(_DW + lvl) + sc(_DB + lvl), 0.0) * masks[lvl]
        downs.append(h)
        s = sigmoid(h * sc(_PW + lvl) + sc(_PB + lvl))
        sel, rank = rank_topk(s, key, masks[lvl], _KS[lvl])
        h = h * s * sel
        key = rank
        invs, tvp = pool_graph(bmats[lvl], bmats[lvl + 1], sel, h)
        invss.append(invs)
        masks.append(sel)

    tv = tvp * invss[4]
    h = jnp.maximum(tv * sc(_BW) + sc(_BB), 0.0) * masks[4]

    for i in range(4):
        up = 3 - i
        if up == 0:
            tv = mv_g(h)
        else:
            tv = mv_b(bmats[up], h) * invss[up]
        h = jnp.maximum(tv * sc(_UW + i) + sc(_UB + i), 0.0) * masks[up]
        h = h + downs[up]

    acc = mv_g(h * sc(_EW0) + org_h * sc(_EW1))
    res = jnp.maximum(acc + sc(_EB), 0.0) * valid0
    out_ref[0] = t(res)


def _mlp_body(x_ref, g1, b1, w1, c1, g2, b2, w2, c2,
              g3, b3, w3, c3, g4, b4, w4, c4, o_ref):
    def bn(x, ga, be):
        m = jnp.mean(x, axis=0, keepdims=True)
        v = jnp.mean((x - m) ** 2, axis=0, keepdims=True)
        return (x - m) / jnp.sqrt(v + 1e-5) * ga[...] + be[...]

    def dot(a, b):
        return lax.dot_general(a, b, _CN, precision=lax.Precision.HIGHEST,
                               preferred_element_type=_F32)

    h = x_ref[...]
    for ga, be, w, c in ((g1, b1, w1, c1), (g2, b2, w2, c2),
                         (g3, b3, w3, c3), (g4, b4, w4, c4)):
        h = jnp.maximum(bn(h, ga, be), 0.0)
        h = dot(h, w[...]) + c[...]
    z = h - jnp.max(h, axis=1, keepdims=True)
    e = jnp.exp(z)
    o_ref[...] = e / jnp.sum(e, axis=1, keepdims=True)


def kernel(g, params):
    b = g.shape[0]
    pad = _N - _ROI
    swp = jnp.pad(params['start_w'].astype(_F32), ((0, pad), (0, 0)))

    vals = []
    for key in ('down_w', 'down_b', 'up_w', 'up_b', 'pool_w', 'pool_b'):
        vals += [params[key][i].reshape(()) for i in range(4)]
    vals += [params['start_b'].reshape(()),
             params['bottom_w'].reshape(()), params['bottom_b'].reshape(()),
             params['end_w'][0, 0], params['end_w'][1, 0],
             params['end_b'].reshape(())]
    scal = jnp.stack([v.astype(_F32) for v in vals])

    unet_out = pl.pallas_call(
        _unet_body,
        grid=(b,),
        in_specs=[
            pl.BlockSpec((1, _ROI, _ROI), lambda i: (i, 0, 0),
                         pipeline_mode=pl.Buffered(buffer_count=2)),
            pl.BlockSpec((_N, 1), lambda i: (0, 0)),
            pl.BlockSpec(memory_space=pltpu.SMEM),
        ],
        out_specs=pl.BlockSpec((1, 1, _N), lambda i: (i, 0, 0)),
        out_shape=jax.ShapeDtypeStruct((b, 1, _N), _F32),
        scratch_shapes=[pltpu.VMEM((_N, _N), _F32)]
        + [pltpu.VMEM((_N, _N), _F8)] * 4
        + [pltpu.VMEM((_N, 1), _F32)] * 6,
    )(g.astype(_F32), swp, scal)
    unet_out = unet_out.reshape(b, _N)

    w1p = jnp.pad(params['fl1_w'].astype(_F32), ((0, pad), (0, 0)))
    g1p = jnp.pad(params['bn1_g'].astype(_F32), (0, pad)).reshape(1, _N)
    b1p = jnp.pad(params['bn1_b'].astype(_F32), (0, pad)).reshape(1, _N)

    args = [unet_out,
            g1p, b1p, w1p, params['fl1_b'].reshape(1, -1),
            params['bn2_g'].reshape(1, -1), params['bn2_b'].reshape(1, -1),
            params['fl2_w'], params['fl2_b'].reshape(1, -1),
            params['bn3_g'].reshape(1, -1), params['bn3_b'].reshape(1, -1),
            params['fl3_w'], params['fl3_b'].reshape(1, -1),
            params['bn4_g'].reshape(1, -1), params['bn4_b'].reshape(1, -1),
            params['fl4_w'], params['fl4_b'].reshape(1, -1)]

    out = pl.pallas_call(
        _mlp_body,
        out_shape=jax.ShapeDtypeStruct((b, 2), _F32),
    )(*[a.astype(_F32) for a in args])
    return out


# speedup vs baseline: 4.4976x; 1.0502x over previous
import jax
import jax.numpy as jnp
from jax import lax
from jax.experimental import pallas as pl
from jax.experimental.pallas import tpu as pltpu

_ROI = 1500
_N = 1536
_S = 256
_NS = _N // _S
_LAST = _NS - 1
_KS = (1200, 900, 600, 300)

_DW, _DB, _UW, _UB, _PW, _PB = 0, 4, 8, 12, 16, 20
_SB, _BW, _BB, _EW0, _EW1, _EB = 24, 25, 26, 27, 28, 29

_F32 = jnp.float32
_BF16 = jnp.bfloat16
_F8 = jnp.float8_e4m3fn
_CN = (((1,), (0,)), ((), ()))


def _unet_body(g_ref, sw_ref, sc_ref, out_ref,
               gp_ref, b0, b1, b2, b3, rank_ref, rs_ref, s_sc, k_sc, v_sc,
               mv_ref):
    bmats = (b0, b1, b2, b3, b0)

    def sc(i):
        return sc_ref[i]

    def t(x):
        return jnp.transpose(x)

    swrow = t(sw_ref[...])

    def strip_gp(i, c):
        gp_ref[pl.ds(i * _S, _S), 0:_ROI] = g_ref[0, pl.ds(i * _S, _S), :]
        gp_ref[pl.ds(i * _S, _S), _ROI:_N] = jnp.zeros((_S, _N - _ROI), _F32)
        gs = gp_ref[pl.ds(i * _S, _S), :]
        b0[pl.ds(i * _S, _S), :] = (gs != 0.0).astype(_F8)
        mv_ref[pl.ds(i * _S, _S), :] = jnp.sum(
            gs * swrow, axis=1, keepdims=True)
        return c

    lax.fori_loop(0, _NS - 1, strip_gp, 0)
    gp_ref[pl.ds(_LAST * _S, _S), :] = jnp.zeros((_S, _N), _F32)
    gp_ref[pl.ds(_LAST * _S, _ROI - _LAST * _S), 0:_ROI] = (
        g_ref[0, pl.ds(_LAST * _S, _ROI - _LAST * _S), :])
    gs_t = gp_ref[pl.ds(_LAST * _S, _S), :]
    b0[pl.ds(_LAST * _S, _S), :] = (gs_t != 0.0).astype(_F8)
    mv_ref[pl.ds(_LAST * _S, _S), :] = jnp.sum(
        gs_t * swrow, axis=1, keepdims=True)

    def mv_g(x):
        xr = t(x)

        def strip(i, c):
            left = gp_ref[pl.ds(i * _S, _S), :]
            mv_ref[pl.ds(i * _S, _S), :] = jnp.sum(
                left * xr, axis=1, keepdims=True)
            return c

        lax.fori_loop(0, _NS, strip, 0, unroll=2)
        return mv_ref[...]

    def mv_b(bref, x):
        xr = t(x)

        def strip(i, cc):
            left = bref[pl.ds(i * _S, _S), :].astype(_F32)
            mv_ref[pl.ds(i * _S, _S), :] = jnp.sum(
                left * xr, axis=1, keepdims=True)
            return cc

        lax.fori_loop(0, _NS, strip, 0, unroll=2)
        return mv_ref[...]

    def sigmoid(x):
        e = jnp.exp(-jnp.abs(x))
        return jnp.where(x >= 0, 1.0 / (1.0 + e), e / (1.0 + e))

    def rank_topk(s, key, valid, k):
        sm = jnp.where(valid > 0.0, s, -1.0)
        srow, krow = t(sm), t(key)
        s_sc[...] = sm
        k_sc[...] = key

        def strip(i, carry):
            sc_ = s_sc[pl.ds(i * _S, _S), :]
            kc_ = k_sc[pl.ds(i * _S, _S), :]
            beats = (srow > sc_) | ((srow == sc_) & (krow < kc_))
            rank_ref[pl.ds(i * _S, _S), :] = jnp.sum(
                jnp.where(beats, 1.0, 0.0), axis=1, keepdims=True)
            return carry

        lax.fori_loop(0, _NS, strip, 0, unroll=2)
        rank = rank_ref[...]
        sel = jnp.where((valid > 0.0) & (rank < float(k)), 1.0, 0.0)
        return sel, rank

    def pool_graph(bsrc, bdst, sel, hvec):
        selrow = t(sel)
        v_sc[...] = sel
        hrow = t(hvec)
        bfull = bsrc[...]

        def strip(i, carry):
            left = bsrc[pl.ds(i * _S, _S), :]
            m = lax.dot_general(left, bfull, _CN,
                                preferred_element_type=_F32)
            selc = v_sc[pl.ds(i * _S, _S), :]
            bn = jnp.where((m > 0.0) & (selc > 0.0) & (selrow > 0.0),
                           1.0, 0.0)
            bdst[pl.ds(i * _S, _S), :] = bn.astype(_F8)
            rs_ref[pl.ds(i * _S, _S), :] = jnp.sum(bn, axis=1, keepdims=True)
            mv_ref[pl.ds(i * _S, _S), :] = jnp.sum(
                bn * hrow, axis=1, keepdims=True)
            return carry

        lax.fori_loop(0, _NS, strip, 0, unroll=2)
        return 1.0 / (rs_ref[...] + 1e-8), mv_ref[...]

    iota_col = lax.broadcasted_iota(jnp.int32, (_N, 1), 0)
    valid0 = jnp.where(iota_col < _ROI, 1.0, 0.0)

    h0 = jnp.maximum(mv_ref[...] + sc(_SB), 0.0) * valid0
    org_h = h0

    masks = [valid0]
    invss = [None]
    downs = []
    h = h0
    key = iota_col.astype(_F32)
    tvp = None
    for lvl in range(4):
        if lvl == 0:
            tv = mv_g(h)
        else:
            tv = tvp * invss[lvl]
        h = jnp.maximum(tv * sc(_DW + lvl) + sc(_DB + lvl), 0.0) * masks[lvl]
        downs.append(h)
        s = sigmoid(h * sc(_PW + lvl) + sc(_PB + lvl))
        sel, rank = rank_topk(s, key, masks[lvl], _KS[lvl])
        h = h * s * sel
        key = rank
        invs, tvp = pool_graph(bmats[lvl], bmats[lvl + 1], sel, h)
        invss.append(invs)
        masks.append(sel)

    tv = tvp * invss[4]
    h = jnp.maximum(tv * sc(_BW) + sc(_BB), 0.0) * masks[4]

    for i in range(4):
        up = 3 - i
        if up == 0:
            tv = mv_g(h)
        else:
            tv = mv_b(bmats[up], h) * invss[up]
        h = jnp.maximum(tv * sc(_UW + i) + sc(_UB + i), 0.0) * masks[up]
        h = h + downs[up]

    acc = mv_g(h * sc(_EW0) + org_h * sc(_EW1))
    res = jnp.maximum(acc + sc(_EB), 0.0) * valid0
    out_ref[0] = t(res)


def _mlp_body(x_ref, g1, b1, w1, c1, g2, b2, w2, c2,
              g3, b3, w3, c3, g4, b4, w4, c4, o_ref):
    def bn(x, ga, be):
        m = jnp.mean(x, axis=0, keepdims=True)
        v = jnp.mean((x - m) ** 2, axis=0, keepdims=True)
        return (x - m) / jnp.sqrt(v + 1e-5) * ga[...] + be[...]

    def dot(a, b):
        return lax.dot_general(a, b, _CN, precision=lax.Precision.HIGHEST,
                               preferred_element_type=_F32)

    h = x_ref[...]
    for ga, be, w, c in ((g1, b1, w1, c1), (g2, b2, w2, c2),
                         (g3, b3, w3, c3), (g4, b4, w4, c4)):
        h = jnp.maximum(bn(h, ga, be), 0.0)
        h = dot(h, w[...]) + c[...]
    z = h - jnp.max(h, axis=1, keepdims=True)
    e = jnp.exp(z)
    o_ref[...] = e / jnp.sum(e, axis=1, keepdims=True)


def kernel(g, params):
    b = g.shape[0]
    pad = _N - _ROI
    swp = jnp.pad(params['start_w'].astype(_F32), ((0, pad), (0, 0)))

    vals = []
    for key in ('down_w', 'down_b', 'up_w', 'up_b', 'pool_w', 'pool_b'):
        vals += [params[key][i].reshape(()) for i in range(4)]
    vals += [params['start_b'].reshape(()),
             params['bottom_w'].reshape(()), params['bottom_b'].reshape(()),
             params['end_w'][0, 0], params['end_w'][1, 0],
             params['end_b'].reshape(())]
    scal = jnp.stack([v.astype(_F32) for v in vals])

    unet_out = pl.pallas_call(
        _unet_body,
        grid=(b,),
        in_specs=[
            pl.BlockSpec((1, _ROI, _ROI), lambda i: (i, 0, 0),
                         pipeline_mode=pl.Buffered(buffer_count=2)),
            pl.BlockSpec((_N, 1), lambda i: (0, 0)),
            pl.BlockSpec(memory_space=pltpu.SMEM),
        ],
        out_specs=pl.BlockSpec((1, 1, _N), lambda i: (i, 0, 0)),
        out_shape=jax.ShapeDtypeStruct((b, 1, _N), _F32),
        scratch_shapes=[pltpu.VMEM((_N, _N), _F32)]
        + [pltpu.VMEM((_N, _N), _F8)] * 4
        + [pltpu.VMEM((_N, 1), _F32)] * 6,
    )(g.astype(_F32), swp, scal)
    unet_out = unet_out.reshape(b, _N)

    w1p = jnp.pad(params['fl1_w'].astype(_F32), ((0, pad), (0, 0)))
    g1p = jnp.pad(params['bn1_g'].astype(_F32), (0, pad)).reshape(1, _N)
    b1p = jnp.pad(params['bn1_b'].astype(_F32), (0, pad)).reshape(1, _N)

    args = [unet_out,
            g1p, b1p, w1p, params['fl1_b'].reshape(1, -1),
            params['bn2_g'].reshape(1, -1), params['bn2_b'].reshape(1, -1),
            params['fl2_w'], params['fl2_b'].reshape(1, -1),
            params['bn3_g'].reshape(1, -1), params['bn3_b'].reshape(1, -1),
            params['fl3_w'], params['fl3_b'].reshape(1, -1),
            params['bn4_g'].reshape(1, -1), params['bn4_b'].reshape(1, -1),
            params['fl4_w'], params['fl4_b'].reshape(1, -1)]

    out = pl.pallas_call(
        _mlp_body,
        out_shape=jax.ShapeDtypeStruct((b, 2), _F32),
    )(*[a.astype(_F32) for a in args])
    return out


# speedup vs baseline: 4.5699x; 1.0161x over previous
import jax
import jax.numpy as jnp
from jax import lax
from jax.experimental import pallas as pl
from jax.experimental.pallas import tpu as pltpu

_ROI = 1500
_N = 1536
_S = 512
_NS = _N // _S
_LAST = _NS - 1
_KS = (1200, 900, 600, 300)

_DW, _DB, _UW, _UB, _PW, _PB = 0, 4, 8, 12, 16, 20
_SB, _BW, _BB, _EW0, _EW1, _EB = 24, 25, 26, 27, 28, 29

_F32 = jnp.float32
_BF16 = jnp.bfloat16
_F8 = jnp.float8_e4m3fn
_CN = (((1,), (0,)), ((), ()))


def _unet_body(g_ref, sw_ref, sc_ref, out_ref,
               gp_ref, b0, b1, b2, b3, rank_ref, rs_ref, s_sc, k_sc, v_sc,
               mv_ref):
    bmats = (b0, b1, b2, b3, b0)

    def sc(i):
        return sc_ref[i]

    def t(x):
        return jnp.transpose(x)

    swrow = t(sw_ref[...])

    def strip_gp(i, c):
        gp_ref[pl.ds(i * _S, _S), 0:_ROI] = g_ref[0, pl.ds(i * _S, _S), :]
        gp_ref[pl.ds(i * _S, _S), _ROI:_N] = jnp.zeros((_S, _N - _ROI), _F32)
        gs = gp_ref[pl.ds(i * _S, _S), :]
        b0[pl.ds(i * _S, _S), :] = (gs != 0.0).astype(_F8)
        mv_ref[pl.ds(i * _S, _S), :] = jnp.sum(
            gs * swrow, axis=1, keepdims=True)
        return c

    lax.fori_loop(0, _NS - 1, strip_gp, 0)
    gp_ref[pl.ds(_LAST * _S, _S), :] = jnp.zeros((_S, _N), _F32)
    gp_ref[pl.ds(_LAST * _S, _ROI - _LAST * _S), 0:_ROI] = (
        g_ref[0, pl.ds(_LAST * _S, _ROI - _LAST * _S), :])
    gs_t = gp_ref[pl.ds(_LAST * _S, _S), :]
    b0[pl.ds(_LAST * _S, _S), :] = (gs_t != 0.0).astype(_F8)
    mv_ref[pl.ds(_LAST * _S, _S), :] = jnp.sum(
        gs_t * swrow, axis=1, keepdims=True)

    def mv_g(x):
        xr = t(x)

        def strip(i, c):
            left = gp_ref[pl.ds(i * _S, _S), :]
            mv_ref[pl.ds(i * _S, _S), :] = jnp.sum(
                left * xr, axis=1, keepdims=True)
            return c

        lax.fori_loop(0, _NS, strip, 0)
        return mv_ref[...]

    def mv_b(bref, x):
        xr = t(x)

        def strip(i, cc):
            left = bref[pl.ds(i * _S, _S), :].astype(_F32)
            mv_ref[pl.ds(i * _S, _S), :] = jnp.sum(
                left * xr, axis=1, keepdims=True)
            return cc

        lax.fori_loop(0, _NS, strip, 0)
        return mv_ref[...]

    def sigmoid(x):
        e = jnp.exp(-jnp.abs(x))
        return jnp.where(x >= 0, 1.0 / (1.0 + e), e / (1.0 + e))

    def rank_topk(s, key, valid, k):
        sm = jnp.where(valid > 0.0, s, -1.0)
        srow, krow = t(sm), t(key)
        s_sc[...] = sm
        k_sc[...] = key

        def strip(i, carry):
            sc_ = s_sc[pl.ds(i * _S, _S), :]
            kc_ = k_sc[pl.ds(i * _S, _S), :]
            beats = (srow > sc_) | ((srow == sc_) & (krow < kc_))
            rank_ref[pl.ds(i * _S, _S), :] = jnp.sum(
                jnp.where(beats, 1.0, 0.0), axis=1, keepdims=True)
            return carry

        lax.fori_loop(0, _NS, strip, 0)
        rank = rank_ref[...]
        sel = jnp.where((valid > 0.0) & (rank < float(k)), 1.0, 0.0)
        return sel, rank

    def pool_graph(bsrc, bdst, sel, hvec):
        selrow = t(sel)
        v_sc[...] = sel
        hrow = t(hvec)
        bfull = bsrc[...]

        def strip(i, carry):
            left = bsrc[pl.ds(i * _S, _S), :]
            m = lax.dot_general(left, bfull, _CN,
                                preferred_element_type=_F32)
            selc = v_sc[pl.ds(i * _S, _S), :]
            bn = jnp.where((m > 0.0) & (selc > 0.0) & (selrow > 0.0),
                           1.0, 0.0)
            bdst[pl.ds(i * _S, _S), :] = bn.astype(_F8)
            rs_ref[pl.ds(i * _S, _S), :] = jnp.sum(bn, axis=1, keepdims=True)
            mv_ref[pl.ds(i * _S, _S), :] = jnp.sum(
                bn * hrow, axis=1, keepdims=True)
            return carry

        lax.fori_loop(0, _NS, strip, 0)
        return 1.0 / (rs_ref[...] + 1e-8), mv_ref[...]

    iota_col = lax.broadcasted_iota(jnp.int32, (_N, 1), 0)
    valid0 = jnp.where(iota_col < _ROI, 1.0, 0.0)

    h0 = jnp.maximum(mv_ref[...] + sc(_SB), 0.0) * valid0
    org_h = h0

    masks = [valid0]
    invss = [None]
    downs = []
    h = h0
    key = iota_col.astype(_F32)
    tvp = None
    for lvl in range(4):
        if lvl == 0:
            tv = mv_g(h)
        else:
            tv = tvp * invss[lvl]
        h = jnp.maximum(tv * sc(_DW + lvl) + sc(_DB + lvl), 0.0) * masks[lvl]
        downs.append(h)
        s = sigmoid(h * sc(_PW + lvl) + sc(_PB + lvl))
        sel, rank = rank_topk(s, key, masks[lvl], _KS[lvl])
        h = h * s * sel
        key = rank
        invs, tvp = pool_graph(bmats[lvl], bmats[lvl + 1], sel, h)
        invss.append(invs)
        masks.append(sel)

    tv = tvp * invss[4]
    h = jnp.maximum(tv * sc(_BW) + sc(_BB), 0.0) * masks[4]

    for i in range(4):
        up = 3 - i
        if up == 0:
            tv = mv_g(h)
        else:
            tv = mv_b(bmats[up], h) * invss[up]
        h = jnp.maximum(tv * sc(_UW + i) + sc(_UB + i), 0.0) * masks[up]
        h = h + downs[up]

    acc = mv_g(h * sc(_EW0) + org_h * sc(_EW1))
    res = jnp.maximum(acc + sc(_EB), 0.0) * valid0
    out_ref[0] = t(res)


def _mlp_body(x_ref, g1, b1, w1, c1, g2, b2, w2, c2,
              g3, b3, w3, c3, g4, b4, w4, c4, o_ref):
    def bn(x, ga, be):
        m = jnp.mean(x, axis=0, keepdims=True)
        v = jnp.mean((x - m) ** 2, axis=0, keepdims=True)
        return (x - m) / jnp.sqrt(v + 1e-5) * ga[...] + be[...]

    def dot(a, b):
        return lax.dot_general(a, b, _CN, precision=lax.Precision.HIGHEST,
                               preferred_element_type=_F32)

    h = x_ref[...]
    for ga, be, w, c in ((g1, b1, w1, c1), (g2, b2, w2, c2),
                         (g3, b3, w3, c3), (g4, b4, w4, c4)):
        h = jnp.maximum(bn(h, ga, be), 0.0)
        h = dot(h, w[...]) + c[...]
    z = h - jnp.max(h, axis=1, keepdims=True)
    e = jnp.exp(z)
    o_ref[...] = e / jnp.sum(e, axis=1, keepdims=True)


def kernel(g, params):
    b = g.shape[0]
    pad = _N - _ROI
    swp = jnp.pad(params['start_w'].astype(_F32), ((0, pad), (0, 0)))

    vals = []
    for key in ('down_w', 'down_b', 'up_w', 'up_b', 'pool_w', 'pool_b'):
        vals += [params[key][i].reshape(()) for i in range(4)]
    vals += [params['start_b'].reshape(()),
             params['bottom_w'].reshape(()), params['bottom_b'].reshape(()),
             params['end_w'][0, 0], params['end_w'][1, 0],
             params['end_b'].reshape(())]
    scal = jnp.stack([v.astype(_F32) for v in vals])

    unet_out = pl.pallas_call(
        _unet_body,
        grid=(b,),
        in_specs=[
            pl.BlockSpec((1, _ROI, _ROI), lambda i: (i, 0, 0),
                         pipeline_mode=pl.Buffered(buffer_count=2)),
            pl.BlockSpec((_N, 1), lambda i: (0, 0)),
            pl.BlockSpec(memory_space=pltpu.SMEM),
        ],
        out_specs=pl.BlockSpec((1, 1, _N), lambda i: (i, 0, 0)),
        out_shape=jax.ShapeDtypeStruct((b, 1, _N), _F32),
        scratch_shapes=[pltpu.VMEM((_N, _N), _F32)]
        + [pltpu.VMEM((_N, _N), _F8)] * 4
        + [pltpu.VMEM((_N, 1), _F32)] * 6,
    )(g.astype(_F32), swp, scal)
    unet_out = unet_out.reshape(b, _N)

    w1p = jnp.pad(params['fl1_w'].astype(_F32), ((0, pad), (0, 0)))
    g1p = jnp.pad(params['bn1_g'].astype(_F32), (0, pad)).reshape(1, _N)
    b1p = jnp.pad(params['bn1_b'].astype(_F32), (0, pad)).reshape(1, _N)

    args = [unet_out,
            g1p, b1p, w1p, params['fl1_b'].reshape(1, -1),
            params['bn2_g'].reshape(1, -1), params['bn2_b'].reshape(1, -1),
            params['fl2_w'], params['fl2_b'].reshape(1, -1),
            params['bn3_g'].reshape(1, -1), params['bn3_b'].reshape(1, -1),
            params['fl3_w'], params['fl3_b'].reshape(1, -1),
            params['bn4_g'].reshape(1, -1), params['bn4_b'].reshape(1, -1),
            params['fl4_w'], params['fl4_b'].reshape(1, -1)]

    out = pl.pallas_call(
        _mlp_body,
        out_shape=jax.ShapeDtypeStruct((b, 2), _F32),
    )(*[a.astype(_F32) for a in args])
    return out


# speedup vs baseline: 4.6179x; 1.0105x over previous
import jax
import jax.numpy as jnp
from jax import lax
from jax.experimental import pallas as pl
from jax.experimental.pallas import tpu as pltpu

_ROI = 1500
_N = 1536
_S = 768
_NS = _N // _S
_LAST = _NS - 1
_KS = (1200, 900, 600, 300)

_DW, _DB, _UW, _UB, _PW, _PB = 0, 4, 8, 12, 16, 20
_SB, _BW, _BB, _EW0, _EW1, _EB = 24, 25, 26, 27, 28, 29

_F32 = jnp.float32
_BF16 = jnp.bfloat16
_F8 = jnp.float8_e4m3fn
_CN = (((1,), (0,)), ((), ()))


def _unet_body(g_ref, sw_ref, sc_ref, out_ref,
               gp_ref, b0, b1, b2, b3, rank_ref, rs_ref, s_sc, k_sc, v_sc,
               mv_ref):
    bmats = (b0, b1, b2, b3, b0)

    def sc(i):
        return sc_ref[i]

    def t(x):
        return jnp.transpose(x)

    swrow = t(sw_ref[...])

    def strip_gp(i, c):
        gp_ref[pl.ds(i * _S, _S), 0:_ROI] = g_ref[0, pl.ds(i * _S, _S), :]
        gp_ref[pl.ds(i * _S, _S), _ROI:_N] = jnp.zeros((_S, _N - _ROI), _F32)
        gs = gp_ref[pl.ds(i * _S, _S), :]
        b0[pl.ds(i * _S, _S), :] = (gs != 0.0).astype(_F8)
        mv_ref[pl.ds(i * _S, _S), :] = jnp.sum(
            gs * swrow, axis=1, keepdims=True)
        return c

    lax.fori_loop(0, _NS - 1, strip_gp, 0)
    gp_ref[pl.ds(_LAST * _S, _S), :] = jnp.zeros((_S, _N), _F32)
    gp_ref[pl.ds(_LAST * _S, _ROI - _LAST * _S), 0:_ROI] = (
        g_ref[0, pl.ds(_LAST * _S, _ROI - _LAST * _S), :])
    gs_t = gp_ref[pl.ds(_LAST * _S, _S), :]
    b0[pl.ds(_LAST * _S, _S), :] = (gs_t != 0.0).astype(_F8)
    mv_ref[pl.ds(_LAST * _S, _S), :] = jnp.sum(
        gs_t * swrow, axis=1, keepdims=True)

    def mv_g(x):
        xr = t(x)

        def strip(i, c):
            left = gp_ref[pl.ds(i * _S, _S), :]
            mv_ref[pl.ds(i * _S, _S), :] = jnp.sum(
                left * xr, axis=1, keepdims=True)
            return c

        lax.fori_loop(0, _NS, strip, 0)
        return mv_ref[...]

    def mv_b(bref, x):
        xr = t(x)

        def strip(i, cc):
            left = bref[pl.ds(i * _S, _S), :].astype(_F32)
            mv_ref[pl.ds(i * _S, _S), :] = jnp.sum(
                left * xr, axis=1, keepdims=True)
            return cc

        lax.fori_loop(0, _NS, strip, 0)
        return mv_ref[...]

    def sigmoid(x):
        e = jnp.exp(-jnp.abs(x))
        return jnp.where(x >= 0, 1.0 / (1.0 + e), e / (1.0 + e))

    def rank_topk(s, key, valid, k):
        sm = jnp.where(valid > 0.0, s, -1.0)
        srow, krow = t(sm), t(key)
        s_sc[...] = sm
        k_sc[...] = key

        def strip(i, carry):
            sc_ = s_sc[pl.ds(i * _S, _S), :]
            kc_ = k_sc[pl.ds(i * _S, _S), :]
            beats = (srow > sc_) | ((srow == sc_) & (krow < kc_))
            rank_ref[pl.ds(i * _S, _S), :] = jnp.sum(
                jnp.where(beats, 1.0, 0.0), axis=1, keepdims=True)
            return carry

        lax.fori_loop(0, _NS, strip, 0)
        rank = rank_ref[...]
        sel = jnp.where((valid > 0.0) & (rank < float(k)), 1.0, 0.0)
        return sel, rank

    def pool_graph(bsrc, bdst, sel, hvec):
        selrow = t(sel)
        v_sc[...] = sel
        hrow = t(hvec)
        bfull = bsrc[...]

        def strip(i, carry):
            left = bsrc[pl.ds(i * _S, _S), :]
            m = lax.dot_general(left, bfull, _CN,
                                preferred_element_type=_F32)
            selc = v_sc[pl.ds(i * _S, _S), :]
            bn = jnp.where((m > 0.0) & (selc > 0.0) & (selrow > 0.0),
                           1.0, 0.0)
            bdst[pl.ds(i * _S, _S), :] = bn.astype(_F8)
            rs_ref[pl.ds(i * _S, _S), :] = jnp.sum(bn, axis=1, keepdims=True)
            mv_ref[pl.ds(i * _S, _S), :] = jnp.sum(
                bn * hrow, axis=1, keepdims=True)
            return carry

        lax.fori_loop(0, _NS, strip, 0)
        return 1.0 / (rs_ref[...] + 1e-8), mv_ref[...]

    iota_col = lax.broadcasted_iota(jnp.int32, (_N, 1), 0)
    valid0 = jnp.where(iota_col < _ROI, 1.0, 0.0)

    h0 = jnp.maximum(mv_ref[...] + sc(_SB), 0.0) * valid0
    org_h = h0

    masks = [valid0]
    invss = [None]
    downs = []
    h = h0
    key = iota_col.astype(_F32)
    tvp = None
    for lvl in range(4):
        if lvl == 0:
            tv = mv_g(h)
        else:
            tv = tvp * invss[lvl]
        h = jnp.maximum(tv * sc(_DW + lvl) + sc(_DB + lvl), 0.0) * masks[lvl]
        downs.append(h)
        s = sigmoid(h * sc(_PW + lvl) + sc(_PB + lvl))
        sel, rank = rank_topk(s, key, masks[lvl], _KS[lvl])
        h = h * s * sel
        key = rank
        invs, tvp = pool_graph(bmats[lvl], bmats[lvl + 1], sel, h)
        invss.append(invs)
        masks.append(sel)

    tv = tvp * invss[4]
    h = jnp.maximum(tv * sc(_BW) + sc(_BB), 0.0) * masks[4]

    for i in range(4):
        up = 3 - i
        if up == 0:
            tv = mv_g(h)
        else:
            tv = mv_b(bmats[up], h) * invss[up]
        h = jnp.maximum(tv * sc(_UW + i) + sc(_UB + i), 0.0) * masks[up]
        h = h + downs[up]

    acc = mv_g(h * sc(_EW0) + org_h * sc(_EW1))
    res = jnp.maximum(acc + sc(_EB), 0.0) * valid0
    out_ref[0] = t(res)


def _mlp_body(x_ref, g1, b1, w1, c1, g2, b2, w2, c2,
              g3, b3, w3, c3, g4, b4, w4, c4, o_ref):
    def bn(x, ga, be):
        m = jnp.mean(x, axis=0, keepdims=True)
        v = jnp.mean((x - m) ** 2, axis=0, keepdims=True)
        return (x - m) / jnp.sqrt(v + 1e-5) * ga[...] + be[...]

    def dot(a, b):
        return lax.dot_general(a, b, _CN, precision=lax.Precision.HIGHEST,
                               preferred_element_type=_F32)

    h = x_ref[...]
    for ga, be, w, c in ((g1, b1, w1, c1), (g2, b2, w2, c2),
                         (g3, b3, w3, c3), (g4, b4, w4, c4)):
        h = jnp.maximum(bn(h, ga, be), 0.0)
        h = dot(h, w[...]) + c[...]
    z = h - jnp.max(h, axis=1, keepdims=True)
    e = jnp.exp(z)
    o_ref[...] = e / jnp.sum(e, axis=1, keepdims=True)


def kernel(g, params):
    b = g.shape[0]
    pad = _N - _ROI
    swp = jnp.pad(params['start_w'].astype(_F32), ((0, pad), (0, 0)))

    vals = []
    for key in ('down_w', 'down_b', 'up_w', 'up_b', 'pool_w', 'pool_b'):
        vals += [params[key][i].reshape(()) for i in range(4)]
    vals += [params['start_b'].reshape(()),
             params['bottom_w'].reshape(()), params['bottom_b'].reshape(()),
             params['end_w'][0, 0], params['end_w'][1, 0],
             params['end_b'].reshape(())]
    scal = jnp.stack([v.astype(_F32) for v in vals])

    unet_out = pl.pallas_call(
        _unet_body,
        grid=(b,),
        in_specs=[
            pl.BlockSpec((1, _ROI, _ROI), lambda i: (i, 0, 0),
                         pipeline_mode=pl.Buffered(buffer_count=2)),
            pl.BlockSpec((_N, 1), lambda i: (0, 0)),
            pl.BlockSpec(memory_space=pltpu.SMEM),
        ],
        out_specs=pl.BlockSpec((1, 1, _N), lambda i: (i, 0, 0)),
        out_shape=jax.ShapeDtypeStruct((b, 1, _N), _F32),
        scratch_shapes=[pltpu.VMEM((_N, _N), _F32)]
        + [pltpu.VMEM((_N, _N), _F8)] * 4
        + [pltpu.VMEM((_N, 1), _F32)] * 6,
    )(g.astype(_F32), swp, scal)
    unet_out = unet_out.reshape(b, _N)

    w1p = jnp.pad(params['fl1_w'].astype(_F32), ((0, pad), (0, 0)))
    g1p = jnp.pad(params['bn1_g'].astype(_F32), (0, pad)).reshape(1, _N)
    b1p = jnp.pad(params['bn1_b'].astype(_F32), (0, pad)).reshape(1, _N)

    args = [unet_out,
            g1p, b1p, w1p, params['fl1_b'].reshape(1, -1),
            params['bn2_g'].reshape(1, -1), params['bn2_b'].reshape(1, -1),
            params['fl2_w'], params['fl2_b'].reshape(1, -1),
            params['bn3_g'].reshape(1, -1), params['bn3_b'].reshape(1, -1),
            params['fl3_w'], params['fl3_b'].reshape(1, -1),
            params['bn4_g'].reshape(1, -1), params['bn4_b'].reshape(1, -1),
            params['fl4_w'], params['fl4_b'].reshape(1, -1)]

    out = pl.pallas_call(
        _mlp_body,
        out_shape=jax.ShapeDtypeStruct((b, 2), _F32),
    )(*[a.astype(_F32) for a in args])
    return out
